# baseline (speedup 1.0000x reference)
import functools

import jax
import jax.numpy as jnp
from jax.experimental import pallas as pl
from jax.experimental.pallas import tpu as pltpu

MAXD = 5
LANES = 128


def _body(nn_ref, dist_ref, texp_ref, out_ref, *, rows, n, h):
    b = pl.program_id(0)
    r = pl.program_id(1)
    nn = nn_ref[b]
    d = dist_ref[0]
    idx = jnp.clip(d, -1, MAXD) + 1
    jio = jax.lax.broadcasted_iota(jnp.int32, (rows, n), 1)
    iio = jax.lax.broadcasted_iota(jnp.int32, (rows, n), 0) + r * rows
    valid = (jio < nn) & (iio < nn)
    idx = jnp.where(valid, idx, 0)

    jper = LANES // h
    ss = jax.lax.broadcasted_iota(jnp.int32, (rows, h, LANES), 1)
    ll = jax.lax.broadcasted_iota(jnp.int32, (rows, h, LANES), 2)
    patt = ss * jper + (ll // h)
    trows = [texp_ref[k, 0:LANES] for k in range(MAXD + 2)]
    for c in range(n // LANES):
        src = jnp.broadcast_to(
            idx[:, None, c * LANES : (c + 1) * LANES], (rows, h, LANES)
        )
        part = jnp.take_along_axis(src, patt, axis=2)
        acc = jnp.zeros((rows, h, LANES), jnp.float32)
        for k in range(1, MAXD + 2):
            acc = jnp.where(part == k, trows[k], acc)
        out_ref[:, c, :, :] = acc


def kernel(dist, batch_num_nodes, embedding_table):
    B, N, _ = dist.shape
    K, H = embedding_table.shape
    texp = jnp.zeros((8, LANES), jnp.float32).at[:K].set(
        jnp.tile(embedding_table, (1, LANES // H))
    )
    ROWS = 256
    grid = (B, N // ROWS)
    C = N // LANES

    out = pl.pallas_call(
        functools.partial(_body, rows=ROWS, n=N, h=H),
        grid_spec=pltpu.PrefetchScalarGridSpec(
            num_scalar_prefetch=1,
            grid=grid,
            in_specs=[
                pl.BlockSpec((1, ROWS, N), lambda b, r, nn: (b, r, 0)),
                pl.BlockSpec((8, LANES), lambda b, r, nn: (0, 0)),
            ],
            out_specs=pl.BlockSpec(
                (ROWS, C, H, LANES),
                lambda b, r, nn: (b * (N // ROWS) + r, 0, 0, 0),
            ),
        ),
        out_shape=jax.ShapeDtypeStruct((B * N, C, H, LANES), jnp.float32),
    )(batch_num_nodes.astype(jnp.int32), dist, texp)
    return out.reshape(B, N, N, H)


# speedup vs baseline: 5.3240x; 5.3240x over previous
import functools

import jax
import jax.numpy as jnp
from jax.experimental import pallas as pl
from jax.experimental.pallas import tpu as pltpu

MAXD = 5


def _body(nn_ref, dist_ref, tc_ref, out_ref, *, rows, n, h):
    b = pl.program_id(0)
    r = pl.program_id(1)
    nn = nn_ref[b]
    d = dist_ref[0]
    idx = jnp.clip(d, -1, MAXD) + 1
    jio = jax.lax.broadcasted_iota(jnp.int32, (rows, n), 1)
    iio = jax.lax.broadcasted_iota(jnp.int32, (rows, n), 0) + r * rows
    valid = (jio < nn) & (iio < nn)
    idx = jnp.where(valid, idx, 0)

    idx8 = jnp.broadcast_to(idx[:, None, :], (rows, h, n))
    acc = jnp.zeros((rows, h, n), jnp.float32)
    for k in range(1, MAXD + 2):
        acc = jnp.where(idx8 == k, tc_ref[k], acc)
    out_ref[0] = acc


def kernel(dist, batch_num_nodes, embedding_table):
    B, N, _ = dist.shape
    K, H = embedding_table.shape
    tc = jnp.broadcast_to(embedding_table[:, :, None], (K, H, N))
    ROWS = 256
    grid = (B, N // ROWS)

    out = pl.pallas_call(
        functools.partial(_body, rows=ROWS, n=N, h=H),
        grid_spec=pltpu.PrefetchScalarGridSpec(
            num_scalar_prefetch=1,
            grid=grid,
            in_specs=[
                pl.BlockSpec((1, ROWS, N), lambda b, r, nn: (b, r, 0)),
                pl.BlockSpec((K, H, N), lambda b, r, nn: (0, 0, 0)),
            ],
            out_specs=pl.BlockSpec(
                (1, ROWS, H, N), lambda b, r, nn: (b, r, 0, 0)
            ),
        ),
        out_shape=jax.ShapeDtypeStruct((B, N, H, N), jnp.float32),
    )(batch_num_nodes.astype(jnp.int32), dist, tc)
    return jnp.transpose(out, (0, 1, 3, 2))


# speedup vs baseline: 5.5655x; 1.0454x over previous
import functools

import jax
import jax.numpy as jnp
from jax.experimental import pallas as pl
from jax.experimental.pallas import tpu as pltpu

MAXD = 5


def _body(nn_ref, dist_ref, tc_ref, tb_ref, out_ref, *, rows, n, h):
    b = pl.program_id(0)
    r = pl.program_id(1)
    nn = nn_ref[b]
    d = dist_ref[0]
    idx = jnp.clip(d, -1, MAXD) + 1
    jio = jax.lax.broadcasted_iota(jnp.int32, (rows, n), 1)
    iio = jax.lax.broadcasted_iota(jnp.int32, (rows, n), 0) + r * rows
    valid = (jio < nn) & (iio < nn)
    idx = jnp.where(valid, idx, 0)

    tsrc = jnp.broadcast_to(tc_ref[0], (rows, h, 128))
    for c in range(n // 128):
        sl = slice(c * 128, (c + 1) * 128)
        idx8 = jnp.broadcast_to(idx[:, None, sl], (rows, h, 128))
        if c == 0:
            val = jnp.take_along_axis(tsrc, idx8, axis=2)
        else:
            val = jnp.zeros((rows, h, 128), jnp.float32)
            for k in range(1, MAXD + 2):
                val = jnp.where(idx8 == k, tb_ref[k], val)
        out_ref[0, :, :, sl] = val


def kernel(dist, batch_num_nodes, embedding_table):
    B, N, _ = dist.shape
    K, H = embedding_table.shape
    tz = embedding_table.at[0].set(0.0)
    tc = jnp.zeros((1, H, 128), jnp.float32).at[0, :, :K].set(tz.T)
    tb = jnp.broadcast_to(embedding_table[:, :, None], (K, H, 128))
    ROWS = 256
    grid = (B, N // ROWS)

    out = pl.pallas_call(
        functools.partial(_body, rows=ROWS, n=N, h=H),
        grid_spec=pltpu.PrefetchScalarGridSpec(
            num_scalar_prefetch=1,
            grid=grid,
            in_specs=[
                pl.BlockSpec((1, ROWS, N), lambda b, r, nn: (b, r, 0)),
                pl.BlockSpec((1, H, 128), lambda b, r, nn: (0, 0, 0)),
                pl.BlockSpec((K, H, 128), lambda b, r, nn: (0, 0, 0)),
            ],
            out_specs=pl.BlockSpec(
                (1, ROWS, H, N), lambda b, r, nn: (b, r, 0, 0)
            ),
        ),
        out_shape=jax.ShapeDtypeStruct((B, N, H, N), jnp.float32),
        compiler_params=pltpu.CompilerParams(
            dimension_semantics=("parallel", "parallel")
        ),
    )(batch_num_nodes.astype(jnp.int32), dist, tc, tb)
    return jnp.transpose(out, (0, 1, 3, 2))


# speedup vs baseline: 5.8227x; 1.0462x over previous
import functools

import jax
import jax.numpy as jnp
from jax.experimental import pallas as pl
from jax.experimental.pallas import tpu as pltpu

MAXD = 5


def _body(nn_ref, dist_ref, tc_ref, tb_ref, out_ref, *, rows, n, h):
    b = pl.program_id(0)
    r = pl.program_id(1)
    nn = nn_ref[b]
    d = dist_ref[0]
    idx = jnp.clip(d, -1, MAXD) + 1
    jio = jax.lax.broadcasted_iota(jnp.int32, (rows, n), 1)
    iio = jax.lax.broadcasted_iota(jnp.int32, (rows, n), 0) + r * rows
    valid = (jio < nn) & (iio < nn)
    idx = jnp.where(valid, idx, 0)

    tsrc = jnp.broadcast_to(tc_ref[0], (rows, h, 128))
    for c in range(n // 128):
        sl = slice(c * 128, (c + 1) * 128)
        idx8 = jnp.broadcast_to(idx[:, None, sl], (rows, h, 128))
        if c == 0:
            val = jnp.take_along_axis(tsrc, idx8, axis=2)
        else:
            val = jnp.zeros((rows, h, 128), jnp.float32)
            for k in range(1, MAXD + 2):
                val = jnp.where(idx8 == k, tb_ref[k], val)
        out_ref[0, :, :, sl] = val


def kernel(dist, batch_num_nodes, embedding_table):
    B, N, _ = dist.shape
    K, H = embedding_table.shape
    tz = embedding_table.at[0].set(0.0)
    tc = jnp.zeros((1, H, 128), jnp.float32).at[0, :, :K].set(tz.T)
    tb = jnp.broadcast_to(embedding_table[:, :, None], (K, H, 128))
    ROWS = 512
    grid = (B, N // ROWS)

    out = pl.pallas_call(
        functools.partial(_body, rows=ROWS, n=N, h=H),
        grid_spec=pltpu.PrefetchScalarGridSpec(
            num_scalar_prefetch=1,
            grid=grid,
            in_specs=[
                pl.BlockSpec((1, ROWS, N), lambda b, r, nn: (b, r, 0)),
                pl.BlockSpec((1, H, 128), lambda b, r, nn: (0, 0, 0)),
                pl.BlockSpec((K, H, 128), lambda b, r, nn: (0, 0, 0)),
            ],
            out_specs=pl.BlockSpec(
                (1, ROWS, H, N), lambda b, r, nn: (b, r, 0, 0)
            ),
        ),
        out_shape=jax.ShapeDtypeStruct((B, N, H, N), jnp.float32),
        compiler_params=pltpu.CompilerParams(
            dimension_semantics=("parallel", "parallel")
        ),
    )(batch_num_nodes.astype(jnp.int32), dist, tc, tb)
    return jnp.transpose(out, (0, 1, 3, 2))
